# Initial kernel scaffold; baseline (speedup 1.0000x reference)
import jax
import jax.numpy as jnp
from jax.experimental import pallas as pl


def kernel(h, e, edge_index, W_emb, b_emb, W_gcn0, b_gcn0, gam0, bet0, W_gcn1, b_gcn1, gam1, bet1, W_gcn2, b_gcn2, gam2, bet2, W_gcn3, b_gcn3, gam3, bet3, W_m0, b_m0, W_m1, b_m1, W_m2, b_m2):
    raise NotImplementedError("write your pallas kernel here")


# speedup vs baseline: 5.6122x; 5.6122x over previous
import functools

import jax
import jax.numpy as jnp
from jax import lax
from jax.experimental import pallas as pl
from jax.experimental.pallas import tpu as pltpu
from jax.experimental.pallas import tpu_sc as plsc

N = 10000
NPAD = 10240
E = 320000
D = 128

NC = 2
NS = 16
NW = NC * NS
EPT = E // NW
K = 80
NCHUNKS = EPT // K
RPT = NPAD // NS


def _mo8(x):
  return pl.multiple_of(x, 8)

_mesh = plsc.VectorSubcoreMesh(core_axis_name="c", subcore_axis_name="s")
_sc_params = pltpu.CompilerParams(use_tc_tiling_on_sc=False)


DH = D // 2
EPT2 = E // NS
NCH2 = EPT2 // K


def _sc_spmm_body(table, gidx4, sidx3, zeros, out, acc, gbuf, sbuf,
                  rows0, rows1, sem0, sem1):
  c = lax.axis_index("c")
  s = lax.axis_index("s")
  row0 = _mo8(s * RPT)
  pltpu.sync_copy(zeros.at[pl.ds(row0, RPT)], acc.at[pl.ds(row0, RPT)])
  pltpu.sync_copy(gidx4.at[c, s], gbuf)
  pltpu.sync_copy(sidx3.at[s], sbuf)
  plsc.subcore_barrier()

  def start(i, rbuf, sem):
    pltpu.async_copy(table.at[gbuf.at[i]], rbuf, sem)

  def wait(i, rbuf, sem):
    pltpu.make_async_copy(table.at[gbuf.at[i]], rbuf, sem).wait()

  start(0, rows0, sem0)

  def body2(j, carry):
    i = j * 2
    start(i + 1, rows1, sem1)
    wait(i, rows0, sem0)
    pltpu.sync_copy(rows0, acc.at[sbuf.at[i]], add=True)
    start(i + 2, rows0, sem0)
    wait(i + 1, rows1, sem1)
    pltpu.sync_copy(rows1, acc.at[sbuf.at[i + 1]], add=True)
    return carry

  lax.fori_loop(0, (NCH2 - 2) // 2, body2, 0)
  start(NCH2 - 1, rows1, sem1)
  wait(NCH2 - 2, rows0, sem0)
  pltpu.sync_copy(rows0, acc.at[sbuf.at[NCH2 - 2]], add=True)
  wait(NCH2 - 1, rows1, sem1)
  pltpu.sync_copy(rows1, acc.at[sbuf.at[NCH2 - 1]], add=True)
  plsc.subcore_barrier()
  pltpu.sync_copy(acc.at[pl.ds(row0, RPT)], out.at[c, pl.ds(row0, RPT)])


_sc_spmm = pl.kernel(
    _sc_spmm_body,
    out_type=jax.ShapeDtypeStruct((NC, NPAD, DH), jnp.float32),
    mesh=_mesh,
    compiler_params=_sc_params,
    scratch_types=[
        pltpu.VMEM_SHARED((NPAD, DH), jnp.float32),
        pltpu.VMEM((NCH2, K), jnp.int32),
        pltpu.VMEM((NCH2, K), jnp.int32),
        pltpu.VMEM((K, DH), jnp.float32),
        pltpu.VMEM((K, DH), jnp.float32),
        pltpu.SemaphoreType.DMA,
        pltpu.SemaphoreType.DMA,
    ],
)


def _sc_degrees_body(sidxS, sidxD, zeros, out, accS, accD, sbufS, sbufD,
                     ones_v, semA, semB):
  c = lax.axis_index("c")
  s = lax.axis_index("s")
  wid = c * NS + s
  row0 = _mo8(s * RPT)
  pltpu.sync_copy(zeros.at[pl.ds(row0, RPT)], accS.at[pl.ds(row0, RPT)])
  pltpu.sync_copy(zeros.at[pl.ds(row0, RPT)], accD.at[pl.ds(row0, RPT)])
  pltpu.sync_copy(sidxS.at[wid], sbufS)
  pltpu.sync_copy(sidxD.at[wid], sbufD)
  ones16 = jnp.full((16,), 1.0, jnp.float32)

  def fill(k, carry):
    ones_v[k] = ones16
    return carry

  lax.fori_loop(0, K, fill, 0)
  plsc.subcore_barrier()

  def body2(i, carry):
    a = pltpu.async_copy(ones_v, accS.at[sbufS.at[i]], semA, add=True)
    b = pltpu.async_copy(ones_v, accD.at[sbufD.at[i]], semB, add=True)
    a.wait()
    b.wait()
    return carry

  lax.fori_loop(0, NCHUNKS, body2, 0)
  plsc.subcore_barrier()
  pltpu.sync_copy(accS.at[pl.ds(row0, RPT)],
                  out.at[c, 0, pl.ds(row0, RPT)])
  pltpu.sync_copy(accD.at[pl.ds(row0, RPT)],
                  out.at[c, 1, pl.ds(row0, RPT)])


_sc_degrees = pl.kernel(
    _sc_degrees_body,
    out_type=jax.ShapeDtypeStruct((NC, 2, NPAD, 16), jnp.float32),
    mesh=_mesh,
    compiler_params=_sc_params,
    scratch_types=[
        pltpu.VMEM_SHARED((NPAD, 16), jnp.float32),
        pltpu.VMEM_SHARED((NPAD, 16), jnp.float32),
        pltpu.VMEM((NCHUNKS, K), jnp.int32),
        pltpu.VMEM((NCHUNKS, K), jnp.int32),
        pltpu.VMEM((K, 16), jnp.float32),
        pltpu.SemaphoreType.DMA,
        pltpu.SemaphoreType.DMA,
    ],
)


def _sc_gather_body(table, gidx3, out, gbuf, rows0, rows1, sem0, sem1):
  c = lax.axis_index("c")
  s = lax.axis_index("s")
  wid = c * NS + s
  base = wid * EPT
  pltpu.sync_copy(gidx3.at[wid], gbuf)

  def start(i, rbuf, sem):
    pltpu.async_copy(table.at[gbuf.at[i]], rbuf, sem)

  def wait(i, rbuf, sem):
    pltpu.make_async_copy(table.at[gbuf.at[i]], rbuf, sem).wait()

  start(0, rows0, sem0)

  def body2(j, carry):
    i = j * 2
    start(i + 1, rows1, sem1)
    wait(i, rows0, sem0)
    pltpu.sync_copy(rows0, out.at[pl.ds(_mo8(base + i * K), K)])
    start(i + 2, rows0, sem0)
    wait(i + 1, rows1, sem1)
    pltpu.sync_copy(rows1, out.at[pl.ds(_mo8(base + (i + 1) * K), K)])
    return carry

  lax.fori_loop(0, (NCHUNKS - 1) // 2, body2, 0)
  wait(NCHUNKS - 1, rows0, sem0)
  pltpu.sync_copy(rows0, out.at[pl.ds(_mo8(base + (NCHUNKS - 1) * K), K)])


_sc_gather = pl.kernel(
    _sc_gather_body,
    out_type=jax.ShapeDtypeStruct((E, D), jnp.float32),
    mesh=_mesh,
    scratch_types=[
        pltpu.VMEM((NCHUNKS, K), jnp.int32),
        pltpu.VMEM((K, D), jnp.float32),
        pltpu.VMEM((K, D), jnp.float32),
        pltpu.SemaphoreType.DMA,
        pltpu.SemaphoreType.DMA,
    ],
)


def _tc_prep_body(degp, h, Wemb, bemb, h1_ref, hn_ref, ninv_ref):
  deg_out = (degp[0, 0] + degp[1, 0])[:N]
  deg_in = (degp[0, 1] + degp[1, 1])[:N]
  ninv_src = lax.rsqrt(jnp.maximum(deg_out, 1.0))
  ninv_dst = lax.rsqrt(jnp.maximum(deg_in, 1.0))
  ninv_ref[0] = ninv_src
  ninv_ref[1] = ninv_dst
  h1 = jnp.dot(h[...], Wemb[...], preferred_element_type=jnp.float32) + bemb[...]
  h1_ref[...] = h1
  hn = h1 * ninv_src[:, :1]
  hn_ref[0] = hn[:, :DH]
  hn_ref[1] = hn[:, DH:]


def _tc_layer_body(aggp, ninv, hprev, W, b, gam, bet, hnext_ref, hnnext_ref):
  agg = jnp.concatenate([aggp[0][:N], aggp[1][:N]], axis=1) * ninv[1][:, :1]
  y = jnp.dot(agg, W[...], preferred_element_type=jnp.float32) + b[...]
  mu = jnp.mean(y, axis=0, keepdims=True)
  dvt = y - mu
  var = jnp.mean(dvt * dvt, axis=0, keepdims=True)
  yn = gam[...] * dvt * lax.rsqrt(var + 1e-5) + bet[...]
  hn = hprev[...] + jnp.maximum(yn, 0.0)
  hnext_ref[...] = hn
  hnn = hn * ninv[0][:, :1]
  hnnext_ref[0] = hnn[:, :DH]
  hnnext_ref[1] = hnn[:, DH:]


def _tc_pq_body(h, Wm0, bm0, p_ref, q_ref):
  p_ref[...] = (jnp.dot(h[...], Wm0[0:D, :], preferred_element_type=jnp.float32)
                + bm0[...])
  q_ref[...] = jnp.dot(h[...], Wm0[D:2 * D, :], preferred_element_type=jnp.float32)


R_BLK = 2560


def _tc_mlp_body(r1, r2, Wm1, bm1, Wm2, bm2, out_ref):
  z = jnp.maximum(r1[...] + r2[...], 0.0)
  z = jnp.maximum(jnp.dot(z, Wm1[...], preferred_element_type=jnp.float32)
                  + bm1[...], 0.0)
  out_ref[...] = jnp.dot(z, Wm2[...], preferred_element_type=jnp.float32) + bm2[...]


_f32 = jnp.float32


def _full(shape):
  return pl.BlockSpec(shape, lambda *_: tuple(0 for _ in shape))


_tc_prep = pl.pallas_call(
    _tc_prep_body,
    out_shape=[jax.ShapeDtypeStruct((N, D), _f32),
               jax.ShapeDtypeStruct((2, N, DH), _f32),
               jax.ShapeDtypeStruct((2, N, 16), _f32)],
)

_tc_layer = pl.pallas_call(
    _tc_layer_body,
    out_shape=[jax.ShapeDtypeStruct((N, D), _f32),
               jax.ShapeDtypeStruct((2, N, DH), _f32)],
)

_tc_pq = pl.pallas_call(
    _tc_pq_body,
    out_shape=[jax.ShapeDtypeStruct((N, D), _f32),
               jax.ShapeDtypeStruct((N, D), _f32)],
)

_tc_mlp = pl.pallas_call(
    _tc_mlp_body,
    grid=(E // R_BLK,),
    in_specs=[
        pl.BlockSpec((R_BLK, D), lambda i: (i, 0)),
        pl.BlockSpec((R_BLK, D), lambda i: (i, 0)),
        pl.BlockSpec((D, 64), lambda i: (0, 0)),
        pl.BlockSpec((1, 64), lambda i: (0, 0)),
        pl.BlockSpec((64, 2), lambda i: (0, 0)),
        pl.BlockSpec((1, 2), lambda i: (0, 0)),
    ],
    out_specs=pl.BlockSpec((R_BLK, 2), lambda i: (i, 0)),
    out_shape=jax.ShapeDtypeStruct((E, 2), _f32),
)

def kernel(h, e, edge_index, W_emb, b_emb, W_gcn0, b_gcn0, gam0, bet0,
           W_gcn1, b_gcn1, gam1, bet1, W_gcn2, b_gcn2, gam2, bet2,
           W_gcn3, b_gcn3, gam3, bet3, W_m0, b_m0, W_m1, b_m1, W_m2, b_m2):
  src3 = edge_index[0].reshape(NW, NCHUNKS, K)
  dst3 = edge_index[1].reshape(NW, NCHUNKS, K)
  src16 = edge_index[0].reshape(NS, NCH2, K)
  gidx4 = jnp.stack([src16, src16 + N])
  sidx3 = edge_index[1].reshape(NS, NCH2, K)
  zeros16 = jnp.zeros((NPAD, 16), _f32)
  zerosH = jnp.zeros((NPAD, DH), _f32)

  degp = _sc_degrees(src3, dst3, zeros16)
  h1, hn2, ninv = _tc_prep(degp, h, W_emb, b_emb.reshape(1, D))

  gcn = [(W_gcn0, b_gcn0, gam0, bet0), (W_gcn1, b_gcn1, gam1, bet1),
         (W_gcn2, b_gcn2, gam2, bet2), (W_gcn3, b_gcn3, gam3, bet3)]
  for (W, b, gam, bet) in gcn:
    aggp = _sc_spmm(hn2.reshape(2 * N, DH), gidx4, sidx3, zerosH)
    h1, hn2 = _tc_layer(aggp, ninv, h1, W, b.reshape(1, D),
                        gam.reshape(1, D), bet.reshape(1, D))

  p, q = _tc_pq(h1, W_m0, b_m0.reshape(1, D))
  r1 = _sc_gather(p, src3)
  r2 = _sc_gather(q, dst3)
  out = _tc_mlp(r1, r2, W_m1, b_m1.reshape(1, 64), W_m2, b_m2.reshape(1, 2))
  return out
